# baseline (speedup 1.0000x reference)
import functools

import jax
import jax.numpy as jnp
from jax import lax
from jax.experimental import pallas as pl
from jax.experimental.pallas import tpu as pltpu
from jax.experimental.pallas import tpu_sc as plsc

_CH = 128
_NB = 2


@functools.lru_cache(maxsize=None)
def _build_gather(n_pad, vocab, d_feat):
    info = plsc.get_sparse_core_info()
    nc, ns = info.num_cores, info.num_subcores
    nw = nc * ns
    b_per_w = n_pad // nw
    nch = b_per_w // _CH
    ngroups = nch // _NB
    assert nch % _NB == 0 and ngroups % 2 == 0
    mesh = plsc.VectorSubcoreMesh(core_axis_name="c", subcore_axis_name="s")

    scratch = [pltpu.VMEM((b_per_w,), jnp.int32)]
    scratch += [pltpu.VMEM((_CH, d_feat), jnp.float32) for _ in range(2 * _NB)]
    scratch += [pltpu.SemaphoreType.DMA] * 4

    half = n_pad // 2
    assert nw % 2 == 0 and half % b_per_w == 0

    @functools.partial(
        pl.kernel,
        mesh=mesh,
        out_type=jax.ShapeDtypeStruct((half, 2 * d_feat), jnp.float32),
        scratch_types=scratch,
        compiler_params=pltpu.CompilerParams(use_tc_tiling_on_sc=False),
    )
    def gather_kernel(table_hbm, idx_hbm, out_hbm, idx_v, *rest):
        bufs = rest[: 2 * _NB]
        gs0, gs1, ws0, ws1 = rest[2 * _NB :]
        set0, set1 = bufs[:_NB], bufs[_NB:]
        wid = lax.axis_index("s") * nc + lax.axis_index("c")
        base = wid * b_per_w
        row0 = lax.rem(base, half)
        col0 = jnp.where(base < half, 0, d_feat)
        pltpu.sync_copy(idx_hbm.at[pl.ds(base, b_per_w)], idx_v)

        def fire_gathers(grp, bufset, sem):
            for b in range(_NB):
                off = (grp * _NB + b) * _CH
                pltpu.async_copy(
                    table_hbm.at[idx_v.at[pl.ds(off, _CH)]], bufset[b], sem
                )

        def drain_gathers(grp, bufset, sem):
            for b in range(_NB):
                off = (grp * _NB + b) * _CH
                pltpu.make_async_copy(
                    table_hbm.at[idx_v.at[pl.ds(off, _CH)]], bufset[b], sem
                ).wait()

        def write_group(grp, bufset, sem):
            for b in range(_NB):
                off = row0 + (grp * _NB + b) * _CH
                pltpu.async_copy(
                    bufset[b],
                    out_hbm.at[pl.ds(off, _CH), pl.ds(col0, d_feat)],
                    sem,
                )
            for b in range(_NB):
                off = row0 + (grp * _NB + b) * _CH
                pltpu.make_async_copy(
                    bufset[b],
                    out_hbm.at[pl.ds(off, _CH), pl.ds(col0, d_feat)],
                    sem,
                ).wait()

        fire_gathers(0, set0, gs0)

        def body(g2, carry):
            i0 = 2 * g2
            fire_gathers(i0 + 1, set1, gs1)
            drain_gathers(i0, set0, gs0)
            write_group(i0, set0, ws0)
            fire_gathers(lax.rem(i0 + 2, ngroups), set0, gs0)
            drain_gathers(i0 + 1, set1, gs1)
            write_group(i0 + 1, set1, ws1)
            return carry

        lax.fori_loop(0, ngroups // 2, body, 0)
        drain_gathers(0, set0, gs0)

    return gather_kernel


def _proj_body(d_feat, emb_ref, w_ref, b_ref, *rest):
    out_ref = rest[-1]
    full = emb_ref[...]
    w = w_ref[...]
    bias = b_ref[...]

    def gelu_proj(x):
        y = jnp.dot(x, w, preferred_element_type=jnp.float32) + bias
        return 0.5 * y * (1.0 + lax.erf(y * 0.7071067811865476))

    out_ref[0, 0] = gelu_proj(full[:, :d_feat])
    out_ref[0, 1] = gelu_proj(full[:, d_feat:])


@functools.lru_cache(maxsize=None)
def _build_project(n_slice, d_feat, d_model, blk, s, n_slices, aliased):
    half = n_slice // 2
    body = functools.partial(_proj_body, d_feat)

    def proj_fn(*ds_body_args):
        in_specs = [
            pl.BlockSpec((blk, 2 * d_feat), lambda i: (i, 0)),
            pl.BlockSpec((d_feat, d_model), lambda i: (0, 0)),
            pl.BlockSpec((1, d_model), lambda i: (0, 0)),
        ]
        kwargs = {}
        if aliased:
            in_specs.append(pl.BlockSpec(memory_space=pl.ANY))
            kwargs["input_output_aliases"] = {3: 0}
        return pl.pallas_call(
            body,
            grid=(half // blk,),
            in_specs=in_specs,
            out_specs=pl.BlockSpec(
                (1, 2, blk, d_model), lambda i: (s, 0, i, 0)
            ),
            out_shape=jax.ShapeDtypeStruct(
                (n_slices, 2, half, d_model), jnp.float32
            ),
            **kwargs,
        )(*ds_body_args)

    return proj_fn


def kernel(input_ids, table, W_proj, b_proj):
    b, l = input_ids.shape
    vocab, d_feat = table.shape
    d_model = W_proj.shape[1]
    n = b * l

    ids = input_ids.reshape(-1).astype(jnp.int32)
    n_slices = 2
    align = 32 * _CH * _NB * 2 * n_slices
    n_pad = ((n + align - 1) // align) * align
    if n_pad != n:
        ids = jnp.concatenate([ids, jnp.zeros((n_pad - n,), jnp.int32)])

    n_slice = n_pad // n_slices
    blk = 8192
    while (n_slice // 2) % blk != 0:
        blk //= 2
    b2 = b_proj.reshape(1, d_model)
    gather = _build_gather(n_slice, vocab, d_feat)
    out = None
    for s in range(n_slices):
        emb_s = gather(table, ids[s * n_slice : (s + 1) * n_slice])
        proj = _build_project(
            n_slice, d_feat, d_model, blk, s, n_slices, aliased=s > 0
        )
        out = proj(emb_s, W_proj, b2) if s == 0 else proj(emb_s, W_proj, b2, out)
    return out.reshape(n_pad, d_model)[:n].reshape(b, l, d_model)


# speedup vs baseline: 1.0091x; 1.0091x over previous
import functools

import jax
import jax.numpy as jnp
from jax import lax
from jax.experimental import pallas as pl
from jax.experimental.pallas import tpu as pltpu
from jax.experimental.pallas import tpu_sc as plsc

_CH = 128
_NB = 4


@functools.lru_cache(maxsize=None)
def _build_gather(n_pad, vocab, d_feat):
    info = plsc.get_sparse_core_info()
    nc, ns = info.num_cores, info.num_subcores
    nw = nc * ns
    b_per_w = n_pad // nw
    nch = b_per_w // _CH
    ngroups = nch // _NB
    assert nch % _NB == 0 and ngroups % 2 == 0
    mesh = plsc.VectorSubcoreMesh(core_axis_name="c", subcore_axis_name="s")

    scratch = [pltpu.VMEM((b_per_w,), jnp.int32)]
    scratch += [pltpu.VMEM((_CH, d_feat), jnp.float32) for _ in range(2 * _NB)]
    scratch += [pltpu.SemaphoreType.DMA] * 4

    half = n_pad // 2
    assert nw % 2 == 0 and half % b_per_w == 0

    @functools.partial(
        pl.kernel,
        mesh=mesh,
        out_type=jax.ShapeDtypeStruct((half, 2 * d_feat), jnp.float32),
        scratch_types=scratch,
        compiler_params=pltpu.CompilerParams(use_tc_tiling_on_sc=False),
    )
    def gather_kernel(table_hbm, idx_hbm, out_hbm, idx_v, *rest):
        bufs = rest[: 2 * _NB]
        gs0, gs1, ws0, ws1 = rest[2 * _NB :]
        set0, set1 = bufs[:_NB], bufs[_NB:]
        wid = lax.axis_index("s") * nc + lax.axis_index("c")
        base = wid * b_per_w
        row0 = lax.rem(base, half)
        col0 = jnp.where(base < half, 0, d_feat)
        pltpu.sync_copy(idx_hbm.at[pl.ds(base, b_per_w)], idx_v)

        def fire_gathers(grp, bufset, sem):
            for b in range(_NB):
                off = (grp * _NB + b) * _CH
                pltpu.async_copy(
                    table_hbm.at[idx_v.at[pl.ds(off, _CH)]], bufset[b], sem
                )

        def drain_gathers(grp, bufset, sem):
            for b in range(_NB):
                off = (grp * _NB + b) * _CH
                pltpu.make_async_copy(
                    table_hbm.at[idx_v.at[pl.ds(off, _CH)]], bufset[b], sem
                ).wait()

        def write_group(grp, bufset, sem):
            for b in range(_NB):
                off = row0 + (grp * _NB + b) * _CH
                pltpu.async_copy(
                    bufset[b],
                    out_hbm.at[pl.ds(off, _CH), pl.ds(col0, d_feat)],
                    sem,
                )
            for b in range(_NB):
                off = row0 + (grp * _NB + b) * _CH
                pltpu.make_async_copy(
                    bufset[b],
                    out_hbm.at[pl.ds(off, _CH), pl.ds(col0, d_feat)],
                    sem,
                ).wait()

        fire_gathers(0, set0, gs0)

        def body(g2, carry):
            i0 = 2 * g2
            fire_gathers(i0 + 1, set1, gs1)
            drain_gathers(i0, set0, gs0)
            write_group(i0, set0, ws0)
            fire_gathers(lax.rem(i0 + 2, ngroups), set0, gs0)
            drain_gathers(i0 + 1, set1, gs1)
            write_group(i0 + 1, set1, ws1)
            return carry

        lax.fori_loop(0, ngroups // 2, body, 0)
        drain_gathers(0, set0, gs0)

    return gather_kernel


def _proj_body(d_feat, emb_ref, w_ref, b_ref, out_ref):
    full = emb_ref[...]
    w = w_ref[...]
    bias = b_ref[...]

    def gelu_proj(x):
        y = jnp.dot(x, w, preferred_element_type=jnp.float32) + bias
        return 0.5 * y * (1.0 + lax.erf(y * 0.7071067811865476))

    out_ref[0] = gelu_proj(full[:, :d_feat])
    out_ref[1] = gelu_proj(full[:, d_feat:])


@functools.lru_cache(maxsize=None)
def _build_project(n_pad, d_feat, d_model, blk):
    half = n_pad // 2
    return pl.pallas_call(
        functools.partial(_proj_body, d_feat),
        grid=(half // blk,),
        in_specs=[
            pl.BlockSpec((blk, 2 * d_feat), lambda i: (i, 0)),
            pl.BlockSpec((d_feat, d_model), lambda i: (0, 0)),
            pl.BlockSpec((1, d_model), lambda i: (0, 0)),
        ],
        out_specs=pl.BlockSpec((2, blk, d_model), lambda i: (0, i, 0)),
        out_shape=jax.ShapeDtypeStruct((2, half, d_model), jnp.float32),
    )


def kernel(input_ids, table, W_proj, b_proj):
    b, l = input_ids.shape
    vocab, d_feat = table.shape
    d_model = W_proj.shape[1]
    n = b * l

    ids = input_ids.reshape(-1).astype(jnp.int32)
    align = 32 * _CH * _NB * 2
    n_pad = ((n + align - 1) // align) * align
    if n_pad != n:
        ids = jnp.concatenate([ids, jnp.zeros((n_pad - n,), jnp.int32)])

    emb = _build_gather(n_pad, vocab, d_feat)(table, ids)

    blk = 8192
    while (n_pad // 2) % blk != 0:
        blk //= 2
    out = _build_project(n_pad, d_feat, d_model, blk)(
        emb, W_proj, b_proj.reshape(1, d_model)
    )
    return out.reshape(n_pad, d_model)[:n].reshape(b, l, d_model)


# speedup vs baseline: 1.0115x; 1.0023x over previous
import functools

import jax
import jax.numpy as jnp
from jax import lax
from jax.experimental import pallas as pl
from jax.experimental.pallas import tpu as pltpu
from jax.experimental.pallas import tpu_sc as plsc

_CH = 128
_NB = 4


@functools.lru_cache(maxsize=None)
def _build_gather(n_pad, vocab, d_feat):
    info = plsc.get_sparse_core_info()
    nc, ns = info.num_cores, info.num_subcores
    nw = nc * ns
    b_per_w = n_pad // nw
    nch = b_per_w // _CH
    ngroups = nch // _NB
    assert nch % _NB == 0 and ngroups % 2 == 0
    mesh = plsc.VectorSubcoreMesh(core_axis_name="c", subcore_axis_name="s")

    scratch = [pltpu.VMEM((b_per_w,), jnp.int32)]
    scratch += [pltpu.VMEM((_CH, d_feat), jnp.float32) for _ in range(2 * _NB)]
    scratch += [pltpu.SemaphoreType.DMA] * 4

    half = n_pad // 2
    assert nw % 2 == 0 and half % b_per_w == 0

    @functools.partial(
        pl.kernel,
        mesh=mesh,
        out_type=jax.ShapeDtypeStruct((half, 2 * d_feat), jnp.float32),
        scratch_types=scratch,
        compiler_params=pltpu.CompilerParams(use_tc_tiling_on_sc=False),
    )
    def gather_kernel(table_hbm, idx_hbm, out_hbm, idx_v, *rest):
        bufs = rest[: 2 * _NB]
        gs0, gs1, ws0, ws1 = rest[2 * _NB :]
        set0, set1 = bufs[:_NB], bufs[_NB:]
        wid = lax.axis_index("s") * nc + lax.axis_index("c")
        base = wid * b_per_w
        row0 = lax.rem(base, half)
        col0 = jnp.where(base < half, 0, d_feat)
        pltpu.sync_copy(idx_hbm.at[pl.ds(base, b_per_w)], idx_v)

        def fire_gathers(grp, bufset, sem):
            for b in range(_NB):
                off = (grp * _NB + b) * _CH
                pltpu.async_copy(
                    table_hbm.at[idx_v.at[pl.ds(off, _CH)]], bufset[b], sem
                )

        def drain_gathers(grp, bufset, sem):
            for b in range(_NB):
                off = (grp * _NB + b) * _CH
                pltpu.make_async_copy(
                    table_hbm.at[idx_v.at[pl.ds(off, _CH)]], bufset[b], sem
                ).wait()

        def write_group(grp, bufset, sem):
            for b in range(_NB):
                off = row0 + (grp * _NB + b) * _CH
                pltpu.async_copy(
                    bufset[b],
                    out_hbm.at[pl.ds(off, _CH), pl.ds(col0, d_feat)],
                    sem,
                )
            for b in range(_NB):
                off = row0 + (grp * _NB + b) * _CH
                pltpu.make_async_copy(
                    bufset[b],
                    out_hbm.at[pl.ds(off, _CH), pl.ds(col0, d_feat)],
                    sem,
                ).wait()

        fire_gathers(0, set0, gs0)

        def body(g2, carry):
            i0 = 2 * g2
            fire_gathers(i0 + 1, set1, gs1)
            drain_gathers(i0, set0, gs0)
            write_group(i0, set0, ws0)
            fire_gathers(lax.rem(i0 + 2, ngroups), set0, gs0)
            drain_gathers(i0 + 1, set1, gs1)
            write_group(i0 + 1, set1, ws1)
            return carry

        lax.fori_loop(0, ngroups // 2, body, 0)
        drain_gathers(0, set0, gs0)

    return gather_kernel


def _proj_body(d_feat, emb_ref, w_ref, b_ref, out_ref):
    full = emb_ref[...]
    w = w_ref[...]
    bias = b_ref[...]

    def gelu_proj(x):
        y = jnp.dot(x, w, preferred_element_type=jnp.float32) + bias
        return 0.5 * y * (1.0 + lax.erf(y * 0.7071067811865476))

    out_ref[0] = gelu_proj(full[:, :d_feat])
    out_ref[1] = gelu_proj(full[:, d_feat:])


@functools.lru_cache(maxsize=None)
def _build_project(n_pad, d_feat, d_model, blk):
    half = n_pad // 2
    return pl.pallas_call(
        functools.partial(_proj_body, d_feat),
        grid=(half // blk,),
        in_specs=[
            pl.BlockSpec((blk, 2 * d_feat), lambda i: (i, 0)),
            pl.BlockSpec((d_feat, d_model), lambda i: (0, 0)),
            pl.BlockSpec((1, d_model), lambda i: (0, 0)),
        ],
        out_specs=pl.BlockSpec((2, blk, d_model), lambda i: (0, i, 0)),
        out_shape=jax.ShapeDtypeStruct((2, half, d_model), jnp.float32),
    )


def kernel(input_ids, table, W_proj, b_proj):
    b, l = input_ids.shape
    vocab, d_feat = table.shape
    d_model = W_proj.shape[1]
    n = b * l

    ids = input_ids.reshape(-1).astype(jnp.int32)
    align = 32 * _CH * _NB * 2
    n_pad = ((n + align - 1) // align) * align
    if n_pad != n:
        ids = jnp.concatenate([ids, jnp.zeros((n_pad - n,), jnp.int32)])

    emb = _build_gather(n_pad, vocab, d_feat)(table, ids)

    blk = 16384
    while (n_pad // 2) % blk != 0:
        blk //= 2
    out = _build_project(n_pad, d_feat, d_model, blk)(
        emb, W_proj, b_proj.reshape(1, d_model)
    )
    return out.reshape(n_pad, d_model)[:n].reshape(b, l, d_model)
